# speedup vs baseline: 4.0867x; 4.0867x over previous
import functools

import jax
import jax.numpy as jnp
from jax import lax
from jax.experimental import pallas as pl
from jax.experimental.pallas import tpu as pltpu
from jax.experimental.pallas import tpu_sc as plsc

E = 320000
D = 128
NC = 2
NS = 16
NW = NC * NS
EPW = E // NW
B = 80
NB = EPW // B
LANES = 16


def _body(x_hbm, src_hbm, dst_hbm, out_hbm,
          src_v, dst_v, a_v, b_v, sem_a, sem_b):
    wid = lax.axis_index("s") * NC + lax.axis_index("c")
    base = wid * EPW

    def block(i, carry):
        off = base + i * B
        pltpu.sync_copy(src_hbm.at[pl.ds(off, B)], src_v)
        pltpu.sync_copy(dst_hbm.at[pl.ds(off, B)], dst_v)
        ca = pltpu.async_copy(x_hbm.at[src_v], a_v, sem_a)
        cb = pltpu.async_copy(x_hbm.at[dst_v], b_v, sem_b)
        ca.wait()
        cb.wait()

        def row(r, c2):
            for c in range(D // LANES):
                sl = pl.ds(c * LANES, LANES)
                a_v[r, sl] = a_v[r, sl] - b_v[r, sl]
            return c2

        lax.fori_loop(0, B, row, 0, unroll=2)
        pltpu.sync_copy(a_v, out_hbm.at[pl.ds(off, B)])
        return carry

    lax.fori_loop(0, NB, block, 0)


@jax.jit
def _sc_gather_sub(x, src, dst):
    mesh = plsc.VectorSubcoreMesh(core_axis_name="c", subcore_axis_name="s")
    return pl.kernel(
        _body,
        out_type=jax.ShapeDtypeStruct((E, D), jnp.float32),
        mesh=mesh,
        scratch_types=[
            pltpu.VMEM((B,), jnp.int32),
            pltpu.VMEM((B,), jnp.int32),
            pltpu.VMEM((B, D), jnp.float32),
            pltpu.VMEM((B, D), jnp.float32),
            pltpu.SemaphoreType.DMA,
            pltpu.SemaphoreType.DMA,
        ],
    )(x, src, dst)


def kernel(x, edge_index):
    ei = edge_index.astype(jnp.int32)
    return _sc_gather_sub(x, ei[0], ei[1])


# speedup vs baseline: 11.7660x; 2.8791x over previous
import jax
import jax.numpy as jnp
from jax import lax
from jax.experimental import pallas as pl
from jax.experimental.pallas import tpu as pltpu
from jax.experimental.pallas import tpu_sc as plsc

E = 320000
D = 128
NC = 2
NS = 16
NW = NC * NS
EPW = E // NW
B = 80
NB = EPW // B
NPAIR = (NB - 3) // 2


def _neg_body(x_ref, o_ref):
    o_ref[...] = -x_ref[...]


def _negate(x):
    return pl.pallas_call(
        _neg_body,
        out_shape=jax.ShapeDtypeStruct(x.shape, x.dtype),
    )(x)


def _body(x_hbm, xn_hbm, src_hbm, dst_hbm, out_hbm,
          si, di, a0, a1, sg0, sg1, ss0, ss1):
    wid = lax.axis_index("s") * NC + lax.axis_index("c")
    base = wid * EPW
    pltpu.sync_copy(src_hbm.at[wid], si)
    pltpu.sync_copy(dst_hbm.at[wid], di)

    a = (a0, a1)
    sg = (sg0, sg1)
    ss = (ss0, ss1)

    def g1_start(g, b):
        pltpu.async_copy(x_hbm.at[si.at[g]], a[b], sg[b])

    def g1_wait(g, b):
        pltpu.make_async_copy(x_hbm.at[si.at[g]], a[b], sg[b]).wait()

    def g2_start(g, b):
        pltpu.async_copy(xn_hbm.at[di.at[g]], a[b], sg[b], add=True)

    def g2_wait(g, b):
        pltpu.make_async_copy(xn_hbm.at[di.at[g]], a[b], sg[b]).wait()

    def s_start(g, b):
        pltpu.async_copy(a[b], out_hbm.at[pl.ds(base + g * B, B)], ss[b])

    def s_wait(g, b):
        pltpu.make_async_copy(a[b], out_hbm.at[pl.ds(base + g * B, B)],
                              ss[b]).wait()

    g1_start(0, 0)
    g1_wait(0, 0)
    g2_start(0, 0)
    g1_start(1, 1)
    g2_wait(0, 0)
    s_start(0, 0)

    def pair(o, carry):
        for b, goff in ((1, 1), (0, 2)):
            g = 2 * o + goff
            g1_wait(g, b)
            g2_start(g, b)
            s_wait(g - 1, 1 - b)
            g1_start(g + 1, 1 - b)
            g2_wait(g, b)
            s_start(g, b)
        return carry

    lax.fori_loop(0, NPAIR, pair, 0)

    gt = 2 * NPAIR + 1
    g1_wait(gt, 1)
    g2_start(gt, 1)
    s_wait(gt - 1, 0)
    g1_start(gt + 1, 0)
    g2_wait(gt, 1)
    s_start(gt, 1)

    g1_wait(NB - 1, 0)
    g2_start(NB - 1, 0)
    s_wait(NB - 2, 1)
    g2_wait(NB - 1, 0)
    s_start(NB - 1, 0)
    s_wait(NB - 1, 0)


@jax.jit
def _sc_gather_sub(x, xneg, src3, dst3):
    mesh = plsc.VectorSubcoreMesh(core_axis_name="c", subcore_axis_name="s")
    return pl.kernel(
        _body,
        out_type=jax.ShapeDtypeStruct((E, D), jnp.float32),
        mesh=mesh,
        scratch_types=[
            pltpu.VMEM((NB, B), jnp.int32),
            pltpu.VMEM((NB, B), jnp.int32),
            pltpu.VMEM((B, D), jnp.float32),
            pltpu.VMEM((B, D), jnp.float32),
            pltpu.SemaphoreType.DMA,
            pltpu.SemaphoreType.DMA,
            pltpu.SemaphoreType.DMA,
            pltpu.SemaphoreType.DMA,
        ],
    )(x, xneg, src3, dst3)


def kernel(x, edge_index):
    ei = edge_index.astype(jnp.int32)
    src3 = ei[0].reshape(NW, NB, B)
    dst3 = ei[1].reshape(NW, NB, B)
    return _sc_gather_sub(x, _negate(x), src3, dst3)


# speedup vs baseline: 13.5246x; 1.1495x over previous
import jax
import jax.numpy as jnp
from jax import lax
from jax.experimental import pallas as pl
from jax.experimental.pallas import tpu as pltpu
from jax.experimental.pallas import tpu_sc as plsc

E = 320000
D = 128
NC = 2
NS = 16
NW = NC * NS
EPW = E // NW
B = 80
NB = EPW // B
NPAIR = (NB - 3) // 2


def _neg_body(x_ref, o_ref):
    o_ref[...] = -x_ref[...]


def _negate(x):
    return pl.pallas_call(
        _neg_body,
        out_shape=jax.ShapeDtypeStruct(x.shape, x.dtype),
    )(x)


N_NODES = 10000
ROWS_PER_TILE = 624


def _body(x_hbm, xn_hbm, src_hbm, dst_hbm, out_hbm,
          si, di0, di1, a0, a1, xs, sg0, sg1, ss0, ss1, sd0, sd1):
    sid = lax.axis_index("s")
    wid = sid * NC + lax.axis_index("c")
    base = wid * EPW

    r0 = sid * ROWS_PER_TILE

    @pl.when(sid < NS - 1)
    def _():
        pltpu.sync_copy(x_hbm.at[pl.ds(r0, ROWS_PER_TILE)],
                        xs.at[pl.ds(r0, ROWS_PER_TILE)])

    @pl.when(sid == NS - 1)
    def _():
        t0 = (NS - 1) * ROWS_PER_TILE
        pltpu.sync_copy(x_hbm.at[pl.ds(t0, N_NODES - t0)],
                        xs.at[pl.ds(t0, N_NODES - t0)])

    pltpu.sync_copy(src_hbm.at[wid], si)
    plsc.subcore_barrier()

    a = (a0, a1)
    di = (di0, di1)
    sg = (sg0, sg1)
    ss = (ss0, ss1)
    sd = (sd0, sd1)

    def i_start(g, b):
        pltpu.async_copy(dst_hbm.at[wid * NB + g], di[b], sd[b])

    def i_wait(g, b):
        pltpu.make_async_copy(dst_hbm.at[wid * NB + g], di[b], sd[b]).wait()

    def g1_start(g, b):
        pltpu.async_copy(xs.at[si.at[g]], a[b], sg[b])

    def g1_wait(g, b):
        pltpu.make_async_copy(xs.at[si.at[g]], a[b], sg[b]).wait()

    def g2_start(g, b):
        pltpu.async_copy(xn_hbm.at[di[b].at[0]], a[b], sg[b], add=True)

    def g2_wait(g, b):
        pltpu.make_async_copy(xn_hbm.at[di[b].at[0]], a[b], sg[b]).wait()

    def s_start(g, b):
        pltpu.async_copy(a[b], out_hbm.at[pl.ds(base + g * B, B)], ss[b])

    def s_wait(g, b):
        pltpu.make_async_copy(a[b], out_hbm.at[pl.ds(base + g * B, B)],
                              ss[b]).wait()

    i_start(0, 0)
    g1_start(0, 0)
    g1_wait(0, 0)
    i_wait(0, 0)
    g2_start(0, 0)
    i_start(1, 1)
    g1_start(1, 1)
    g2_wait(0, 0)
    s_start(0, 0)

    def pair(o, carry):
        for b, goff in ((1, 1), (0, 2)):
            g = 2 * o + goff
            g1_wait(g, b)
            i_wait(g, b)
            g2_start(g, b)
            i_start(g + 1, 1 - b)
            s_wait(g - 1, 1 - b)
            g1_start(g + 1, 1 - b)
            g2_wait(g, b)
            s_start(g, b)
        return carry

    lax.fori_loop(0, NPAIR, pair, 0)

    gt = 2 * NPAIR + 1
    g1_wait(gt, 1)
    i_wait(gt, 1)
    g2_start(gt, 1)
    i_start(NB - 1, 0)
    s_wait(gt - 1, 0)
    g1_start(NB - 1, 0)
    g2_wait(gt, 1)
    s_start(gt, 1)

    g1_wait(NB - 1, 0)
    i_wait(NB - 1, 0)
    g2_start(NB - 1, 0)
    s_wait(NB - 2, 1)
    g2_wait(NB - 1, 0)
    s_start(NB - 1, 0)
    s_wait(NB - 1, 0)


@jax.jit
def _sc_gather_sub(x, xneg, src3, dst3):
    mesh = plsc.VectorSubcoreMesh(core_axis_name="c", subcore_axis_name="s")
    return pl.kernel(
        _body,
        out_type=jax.ShapeDtypeStruct((E, D), jnp.float32),
        mesh=mesh,
        scratch_types=[
            pltpu.VMEM((NB, B), jnp.int32),
            pltpu.VMEM((1, B), jnp.int32),
            pltpu.VMEM((1, B), jnp.int32),
            pltpu.VMEM((B, D), jnp.float32),
            pltpu.VMEM((B, D), jnp.float32),
            pltpu.VMEM_SHARED((N_NODES, D), jnp.float32),
            pltpu.SemaphoreType.DMA,
            pltpu.SemaphoreType.DMA,
            pltpu.SemaphoreType.DMA,
            pltpu.SemaphoreType.DMA,
            pltpu.SemaphoreType.DMA,
            pltpu.SemaphoreType.DMA,
        ],
    )(x, xneg, src3, dst3)


def kernel(x, edge_index):
    ei = edge_index.astype(jnp.int32)
    src3 = ei[0].reshape(NW, NB, B)
    dst3 = ei[1].reshape(NW * NB, 1, B)
    return _sc_gather_sub(x, _negate(x), src3, dst3)
